# baseline (speedup 1.0000x reference)
import functools

import jax
import jax.numpy as jnp
from jax import lax
from jax.experimental import pallas as pl
from jax.experimental.pallas import tpu as pltpu
from jax.experimental.pallas import tpu_sc as plsc

D = 64
DP = 128
NC, NS = 2, 16
NW = NC * NS


@functools.lru_cache(maxsize=None)
def _make_gather(batch: int, hist: int):
    rows_per_w = batch // NW
    mesh = plsc.VectorSubcoreMesh(core_axis_name="c", subcore_axis_name="s")

    @functools.partial(
        pl.kernel,
        mesh=mesh,
        out_type=jax.ShapeDtypeStruct((batch * hist, DP), jnp.float32),
        scratch_types=[
            pltpu.VMEM((rows_per_w, hist), jnp.int32),
            pltpu.VMEM((hist, D), jnp.float32),
            pltpu.SemaphoreType.DMA,
        ],
        compiler_params=pltpu.CompilerParams(use_tc_tiling_on_sc=False),
    )
    def gather_kernel(table_hbm, idx_hbm, out_hbm, idx_v, rows_v, sem):
        wid = lax.axis_index("s") * NC + lax.axis_index("c")
        first_row = wid * rows_per_w
        pltpu.sync_copy(idx_hbm.at[pl.ds(first_row, rows_per_w), :], idx_v)

        def body(i, carry):
            pltpu.async_copy(table_hbm.at[idx_v.at[i]], rows_v, sem).wait()
            pltpu.sync_copy(
                rows_v,
                out_hbm.at[pl.ds((first_row + i) * hist, hist), pl.ds(0, D)],
            )
            return carry

        lax.fori_loop(0, rows_per_w, body, 0)

    return gather_kernel


def kernel(glove_embedding_matrix, inputs):
    batch, hist = inputs.shape
    idx = inputs.astype(jnp.int32)
    out = _make_gather(batch, hist)(glove_embedding_matrix, idx)
    return out[:, :D].reshape(batch, hist, D)
